# Initial kernel scaffold; baseline (speedup 1.0000x reference)
import jax
import jax.numpy as jnp
from jax.experimental import pallas as pl


def kernel(x, edge_index, W_l1, b_l1, W_r1, W_l2, b_l2, W_r2):
    raise NotImplementedError("write your pallas kernel here")


# speedup vs baseline: 6.1573x; 6.1573x over previous
import functools

import jax
import jax.numpy as jnp
from jax import lax
from jax.experimental import pallas as pl
from jax.experimental.pallas import tpu as pltpu
from jax.experimental.pallas import tpu_sc as plsc

NC = 2
NS = 16
LN = 16


def _make_sc_agg(N, E, Dw):
    NW = NC * NS
    e_per_w = E // NW
    CH = 128
    n_full = e_per_w // CH
    tail = e_per_w - n_full * CH
    Np = -(-N // (NS * 8)) * (NS * 8)
    rows_per_tile = Np // NS
    assert E % NW == 0 and tail % LN == 0

    mesh = plsc.VectorSubcoreMesh(core_axis_name="c", subcore_axis_name="s")

    @functools.partial(
        pl.kernel,
        mesh=mesh,
        out_type=jax.ShapeDtypeStruct((NC, Np, Dw), jnp.float32),
        compiler_params=pltpu.CompilerParams(use_tc_tiling_on_sc=False),
        scratch_types=[
            pltpu.VMEM((e_per_w,), jnp.int32),
            pltpu.VMEM((e_per_w,), jnp.int32),
            pltpu.VMEM((CH, Dw), jnp.float32),
            pltpu.VMEM_SHARED((Np, Dw), jnp.float32),
            pltpu.SemaphoreType.DMA,
        ],
    )
    def agg(xa_hbm, src_hbm, dst_hbm, zeros_hbm, out_hbm, src_v, dst_v, rows_v,
            accum_sh, sem):
        c = lax.axis_index("c")
        s = lax.axis_index("s")
        wid = s * NC + c
        row0 = s * rows_per_tile
        pltpu.sync_copy(zeros_hbm.at[pl.ds(row0, rows_per_tile)],
                        accum_sh.at[pl.ds(row0, rows_per_tile)])
        plsc.subcore_barrier()

        base = wid * e_per_w
        pltpu.sync_copy(src_hbm.at[pl.ds(base, e_per_w)], src_v)
        pltpu.sync_copy(dst_hbm.at[pl.ds(base, e_per_w)], dst_v)

        def scatter16(off):
            idx16 = dst_v[pl.ds(off, LN)]
            pltpu.sync_copy(rows_v.at[pl.ds(off % CH, LN)],
                            accum_sh.at[idx16], add=True)

        def chunk_body(j, carry):
            off = pl.multiple_of(j * CH, CH)
            pltpu.async_copy(xa_hbm.at[src_v.at[pl.ds(off, CH)]], rows_v,
                             sem).wait()

            def sub_body(k, carry2):
                scatter16(off + k * LN)
                return carry2

            lax.fori_loop(0, CH // LN, sub_body, 0, unroll=True)
            return carry

        lax.fori_loop(0, n_full, chunk_body, 0)

        if tail:
            toff = n_full * CH
            idx_t = src_v[pl.ds(toff, tail)]
            pltpu.async_copy(xa_hbm.at[idx_t], rows_v.at[pl.ds(0, tail)],
                             sem).wait()
            for k in range(tail // LN):
                idx16 = dst_v[pl.ds(toff + k * LN, LN)]
                pltpu.sync_copy(rows_v.at[pl.ds(k * LN, LN)],
                                accum_sh.at[idx16], add=True)

        plsc.subcore_barrier()
        pltpu.sync_copy(accum_sh.at[pl.ds(row0, rows_per_tile)],
                        out_hbm.at[c, pl.ds(row0, rows_per_tile)])

    return agg


def _make_tc_layer(N, Np, D, H, relu, emit_ones, R=1000):
    assert N % R == 0
    Do = H + LN if emit_ones else H

    def body(sums_ref, x_ref, wlt_ref, bl_ref, wrt_ref, o_ref):
        s = sums_ref[0] + sums_ref[1]
        cnt = jnp.sum(s[:, D:], axis=1, keepdims=True) * (1.0 / LN)
        cnt = jnp.maximum(cnt, 1.0)
        agg = s[:, :D] / cnt
        x = x_ref[...][:, :D]
        out = (jnp.dot(agg, wlt_ref[...], preferred_element_type=jnp.float32)
               + bl_ref[...]
               + jnp.dot(x, wrt_ref[...], preferred_element_type=jnp.float32))
        if relu:
            out = jnp.maximum(out, 0.0)
        if emit_ones:
            out = jnp.concatenate(
                [out, jnp.ones((R, LN), dtype=jnp.float32)], axis=1)
        o_ref[...] = out

    Dx = D + LN
    return pl.pallas_call(
        body,
        grid=(N // R,),
        in_specs=[
            pl.BlockSpec((NC, R, D + LN), lambda i: (0, i, 0)),
            pl.BlockSpec((R, Dx), lambda i: (i, 0)),
            pl.BlockSpec((D, H), lambda i: (0, 0)),
            pl.BlockSpec((1, H), lambda i: (0, 0)),
            pl.BlockSpec((D, H), lambda i: (0, 0)),
        ],
        out_specs=pl.BlockSpec((R, Do), lambda i: (i, 0)),
        out_shape=jax.ShapeDtypeStruct((N, Do), jnp.float32),
    )


def _pad_rows(a, Np):
    return jnp.pad(a, ((0, Np - a.shape[0]), (0, 0)))


@jax.jit
def kernel(x, edge_index, W_l1, b_l1, W_r1, W_l2, b_l2, W_r2):
    N, D = x.shape
    E = edge_index.shape[1]
    H = W_l1.shape[0]
    src = edge_index[0]
    dst = edge_index[1]
    Np = -(-N // (NS * 8)) * (NS * 8)
    ones = jnp.ones((N, LN), dtype=jnp.float32)
    xa = jnp.concatenate([x, ones], axis=1)
    zeros = jnp.zeros((Np, D + LN), dtype=jnp.float32)

    sc_agg = _make_sc_agg(N, E, D + LN)
    tc1 = _make_tc_layer(N, Np, D, H, relu=True, emit_ones=True)
    tc2 = _make_tc_layer(N, Np, H, H, relu=False, emit_ones=False)

    sums1 = sc_agg(xa, src, dst, zeros)
    h1a = tc1(sums1, xa, W_l1.T, b_l1[None, :], W_r1.T)
    sums2 = sc_agg(h1a, src, dst, zeros)
    out = tc2(sums2, h1a, W_l2.T, b_l2[None, :], W_r2.T)
    return out


# speedup vs baseline: 6.9995x; 1.1368x over previous
import functools

import jax
import jax.numpy as jnp
from jax import lax
from jax.experimental import pallas as pl
from jax.experimental.pallas import tpu as pltpu
from jax.experimental.pallas import tpu_sc as plsc

NC = 2
NS = 16
LN = 16
CH = 64


def _pad_geometry(N, E):
    NW = NC * NS
    Np = -(-N // (NS * 8)) * (NS * 8)
    n_ch = -(-E // (NW * 2 * CH)) * 2
    return NW, Np, n_ch


def _make_sc_agg(N, E, Dw):
    NW, Np, n_ch = _pad_geometry(N, E)
    rows_per_tile = Np // NS

    mesh = plsc.VectorSubcoreMesh(core_axis_name="c", subcore_axis_name="s")

    @functools.partial(
        pl.kernel,
        mesh=mesh,
        out_type=jax.ShapeDtypeStruct((NC, Np, Dw), jnp.float32),
        compiler_params=pltpu.CompilerParams(use_tc_tiling_on_sc=False),
        scratch_types=[
            pltpu.VMEM((n_ch, CH), jnp.int32),
            pltpu.VMEM((n_ch, CH), jnp.int32),
            pltpu.VMEM((CH, Dw), jnp.float32),
            pltpu.VMEM((CH, Dw), jnp.float32),
            pltpu.VMEM_SHARED((Np, Dw), jnp.float32),
            pltpu.SemaphoreType.DMA,
            pltpu.SemaphoreType.DMA,
        ],
    )
    def agg(xa_hbm, src_hbm, dst_hbm, zeros_hbm, out_hbm, src_v, dst_v,
            rows_a, rows_b, accum_sh, sem_a, sem_b):
        c = lax.axis_index("c")
        s = lax.axis_index("s")
        wid = s * NC + c
        row0 = s * rows_per_tile
        pltpu.sync_copy(zeros_hbm.at[pl.ds(row0, rows_per_tile)],
                        accum_sh.at[pl.ds(row0, rows_per_tile)])
        plsc.subcore_barrier()

        pltpu.sync_copy(src_hbm.at[wid], src_v)
        pltpu.sync_copy(dst_hbm.at[wid], dst_v)

        def start_gather(j, buf, sem):
            pltpu.async_copy(xa_hbm.at[src_v.at[j]], buf, sem)

        def wait_gather(buf, sem):
            pltpu.make_async_copy(xa_hbm.at[pl.ds(0, CH)], buf, sem).wait()

        def scatter(j, buf):
            pltpu.sync_copy(buf, accum_sh.at[dst_v.at[j]], add=True)

        start_gather(0, rows_a, sem_a)

        def pair_body(i, carry):
            j = i * 2
            wait_gather(rows_a, sem_a)
            start_gather(j + 1, rows_b, sem_b)
            scatter(j, rows_a)
            wait_gather(rows_b, sem_b)

            @pl.when(j + 2 < n_ch)
            def _():
                start_gather(j + 2, rows_a, sem_a)

            scatter(j + 1, rows_b)
            return carry

        lax.fori_loop(0, n_ch // 2, pair_body, 0)

        plsc.subcore_barrier()
        pltpu.sync_copy(accum_sh.at[pl.ds(row0, rows_per_tile)],
                        out_hbm.at[c, pl.ds(row0, rows_per_tile)])

    return agg


def _make_tc_layer(N, D, H, relu, emit_ones, R=1000):
    assert N % R == 0
    Do = H + LN if emit_ones else H

    def body(sums_ref, x_ref, wlt_ref, bl_ref, wrt_ref, o_ref):
        s = sums_ref[0] + sums_ref[1]
        cnt = jnp.sum(s[:, D:], axis=1, keepdims=True) * (1.0 / LN)
        cnt = jnp.maximum(cnt, 1.0)
        agg = s[:, :D] / cnt
        x = x_ref[...][:, :D]
        out = (jnp.dot(agg, wlt_ref[...], preferred_element_type=jnp.float32)
               + bl_ref[...]
               + jnp.dot(x, wrt_ref[...], preferred_element_type=jnp.float32))
        if relu:
            out = jnp.maximum(out, 0.0)
        if emit_ones:
            out = jnp.concatenate(
                [out, jnp.ones((R, LN), dtype=jnp.float32)], axis=1)
        o_ref[...] = out

    Dx = D + LN
    return pl.pallas_call(
        body,
        grid=(N // R,),
        in_specs=[
            pl.BlockSpec((NC, R, D + LN), lambda i: (0, i, 0)),
            pl.BlockSpec((R, Dx), lambda i: (i, 0)),
            pl.BlockSpec((D, H), lambda i: (0, 0)),
            pl.BlockSpec((1, H), lambda i: (0, 0)),
            pl.BlockSpec((D, H), lambda i: (0, 0)),
        ],
        out_specs=pl.BlockSpec((R, Do), lambda i: (i, 0)),
        out_shape=jax.ShapeDtypeStruct((N, Do), jnp.float32),
    )


@jax.jit
def kernel(x, edge_index, W_l1, b_l1, W_r1, W_l2, b_l2, W_r2):
    N, D = x.shape
    E = edge_index.shape[1]
    H = W_l1.shape[0]
    NW, Np, n_ch = _pad_geometry(N, E)
    Ep = NW * n_ch * CH
    n_pad = Ep - E
    pad_src = jnp.arange(n_pad, dtype=jnp.int32) % N
    pad_dst = N + jnp.arange(n_pad, dtype=jnp.int32) % (Np - N)
    src = jnp.concatenate([edge_index[0], pad_src]).reshape(NW, n_ch, CH)
    dst = jnp.concatenate([edge_index[1], pad_dst]).reshape(NW, n_ch, CH)

    ones = jnp.ones((N, LN), dtype=jnp.float32)
    xa = jnp.concatenate([x, ones], axis=1)
    zeros = jnp.zeros((Np, D + LN), dtype=jnp.float32)

    sc_agg = _make_sc_agg(N, E, D + LN)
    tc1 = _make_tc_layer(N, D, H, relu=True, emit_ones=True)
    tc2 = _make_tc_layer(N, H, H, relu=False, emit_ones=False)

    sums1 = sc_agg(xa, src, dst, zeros)
    h1a = tc1(sums1, xa, W_l1.T, b_l1[None, :], W_r1.T)
    sums2 = sc_agg(h1a, src, dst, zeros)
    out = tc2(sums2, h1a, W_l2.T, b_l2[None, :], W_r2.T)
    return out


# speedup vs baseline: 8.3006x; 1.1859x over previous
import functools

import jax
import jax.numpy as jnp
from jax import lax
from jax.experimental import pallas as pl
from jax.experimental.pallas import tpu as pltpu
from jax.experimental.pallas import tpu_sc as plsc

NC = 2
NS = 16
LN = 16
CH = 64
BLK = 8
RING = 4


def _pad_geometry(N, E):
    NW = NC * NS
    Np = -(-N // (NS * 8)) * (NS * 8)
    n_ch = -(-(-(-E // NW) // CH) // (2 * BLK)) * 2 * BLK
    return NW, Np, n_ch


def _make_sc_agg(N, E, Dw):
    NW, Np, n_ch = _pad_geometry(N, E)
    n_blk = n_ch // BLK
    pairs = n_blk // 2
    rows_per_tile = Np // NS

    mesh = plsc.VectorSubcoreMesh(core_axis_name="c", subcore_axis_name="s")

    @functools.partial(
        pl.kernel,
        mesh=mesh,
        out_type=jax.ShapeDtypeStruct((NC, Np, Dw), jnp.float32),
        compiler_params=pltpu.CompilerParams(use_tc_tiling_on_sc=False),
        scratch_types=[
            pltpu.VMEM((2, BLK, CH), jnp.int32),
            pltpu.VMEM((2, BLK, CH), jnp.int32),
            pltpu.VMEM((CH, Dw), jnp.float32),
            pltpu.VMEM((CH, Dw), jnp.float32),
            pltpu.VMEM((CH, Dw), jnp.float32),
            pltpu.VMEM((CH, Dw), jnp.float32),
            pltpu.VMEM_SHARED((Np, Dw), jnp.float32),
            pltpu.SemaphoreType.DMA,
            pltpu.SemaphoreType.DMA,
            pltpu.SemaphoreType.DMA,
            pltpu.SemaphoreType.DMA,
            pltpu.SemaphoreType.DMA,
            pltpu.SemaphoreType.DMA,
            pltpu.SemaphoreType.DMA,
            pltpu.SemaphoreType.DMA,
            pltpu.SemaphoreType.DMA,
            pltpu.SemaphoreType.DMA,
        ],
    )
    def agg(xa_hbm, idx_hbm, zeros_hbm, out_hbm, ib0, ib1, r0, r1, r2, r3,
            accum_sh, gs0, gs1, gs2, gs3, ss0, ss1, ss2, ss3, is0, is1):
        c = lax.axis_index("c")
        s = lax.axis_index("s")
        wid = s * NC + c
        row0 = s * rows_per_tile
        rows = [r0, r1, r2, r3]
        gsem = [gs0, gs1, gs2, gs3]
        ssem = [ss0, ss1, ss2, ss3]

        pltpu.sync_copy(zeros_hbm.at[pl.ds(row0, rows_per_tile)],
                        accum_sh.at[pl.ds(row0, rows_per_tile)])
        plsc.subcore_barrier()

        def g_start(ib, k, sl):
            pltpu.async_copy(xa_hbm.at[ib.at[0, k]], rows[sl], gsem[sl])

        def g_wait(sl):
            pltpu.make_async_copy(xa_hbm.at[pl.ds(0, CH)], rows[sl],
                                  gsem[sl]).wait()

        def s_start(ib, k, sl):
            pltpu.async_copy(rows[sl], accum_sh.at[ib.at[1, k]], ssem[sl],
                             add=True)

        def s_wait(sl):
            pltpu.make_async_copy(xa_hbm.at[pl.ds(0, CH)], rows[sl],
                                  ssem[sl]).wait()

        def i_start(b, ib, sem):
            pltpu.async_copy(idx_hbm.at[wid, b], ib, sem)

        def i_wait(ib, sem):
            pltpu.make_async_copy(idx_hbm.at[wid, 0], ib, sem).wait()

        def block(ib, waits_in, drain_out):
            for k in range(RING):
                if waits_in:
                    s_wait(k)
                g_start(ib, k, k)
            for k in range(RING):
                g_wait(k)
                s_start(ib, k, k)
            for k in range(RING):
                s_wait(k)
                g_start(ib, k + RING, k)
            for k in range(RING):
                g_wait(k)
                s_start(ib, k + RING, k)
            if drain_out:
                for k in range(RING):
                    s_wait(k)

        i_start(0, ib0, is0)
        i_start(1, ib1, is1)

        def pair_body(p, carry):
            i_wait(ib0, is0)
            block(ib0, waits_in=False, drain_out=False)
            i_wait(ib1, is1)
            block(ib1, waits_in=True, drain_out=True)

            @pl.when(p < pairs - 1)
            def _():
                i_start(2 * p + 2, ib0, is0)
                i_start(2 * p + 3, ib1, is1)

            return carry

        lax.fori_loop(0, pairs, pair_body, 0)

        plsc.subcore_barrier()
        pltpu.sync_copy(accum_sh.at[pl.ds(row0, rows_per_tile)],
                        out_hbm.at[c, pl.ds(row0, rows_per_tile)])

    return agg


def _make_tc_layer(N, D, H, relu, emit_ones, R=1000):
    assert N % R == 0
    Do = H + LN if emit_ones else H

    def body(sums_ref, x_ref, wlt_ref, bl_ref, wrt_ref, o_ref):
        s = sums_ref[0] + sums_ref[1]
        cnt = jnp.sum(s[:, D:], axis=1, keepdims=True) * (1.0 / LN)
        cnt = jnp.maximum(cnt, 1.0)
        agg = s[:, :D] / cnt
        x = x_ref[...][:, :D]
        out = (jnp.dot(agg, wlt_ref[...], preferred_element_type=jnp.float32)
               + bl_ref[...]
               + jnp.dot(x, wrt_ref[...], preferred_element_type=jnp.float32))
        if relu:
            out = jnp.maximum(out, 0.0)
        if emit_ones:
            out = jnp.concatenate(
                [out, jnp.ones((R, LN), dtype=jnp.float32)], axis=1)
        o_ref[...] = out

    Dx = D + LN
    return pl.pallas_call(
        body,
        grid=(N // R,),
        in_specs=[
            pl.BlockSpec((NC, R, D + LN), lambda i: (0, i, 0)),
            pl.BlockSpec((R, Dx), lambda i: (i, 0)),
            pl.BlockSpec((D, H), lambda i: (0, 0)),
            pl.BlockSpec((1, H), lambda i: (0, 0)),
            pl.BlockSpec((D, H), lambda i: (0, 0)),
        ],
        out_specs=pl.BlockSpec((R, Do), lambda i: (i, 0)),
        out_shape=jax.ShapeDtypeStruct((N, Do), jnp.float32),
    )


@jax.jit
def kernel(x, edge_index, W_l1, b_l1, W_r1, W_l2, b_l2, W_r2):
    N, D = x.shape
    E = edge_index.shape[1]
    H = W_l1.shape[0]
    NW, Np, n_ch = _pad_geometry(N, E)
    n_blk = n_ch // BLK
    Ep = NW * n_ch * CH
    n_pad = Ep - E
    pad_src = jnp.arange(n_pad, dtype=jnp.int32) % N
    pad_dst = N + jnp.arange(n_pad, dtype=jnp.int32) % (Np - N)
    src = jnp.concatenate([edge_index[0], pad_src]).reshape(NW, n_blk, BLK, CH)
    dst = jnp.concatenate([edge_index[1], pad_dst]).reshape(NW, n_blk, BLK, CH)
    idx = jnp.stack([src, dst], axis=2)

    ones = jnp.ones((N, LN), dtype=jnp.float32)
    xa = jnp.concatenate([x, ones], axis=1)
    zeros = jnp.zeros((Np, D + LN), dtype=jnp.float32)

    sc_agg = _make_sc_agg(N, E, D + LN)
    tc1 = _make_tc_layer(N, D, H, relu=True, emit_ones=True)
    tc2 = _make_tc_layer(N, H, H, relu=False, emit_ones=False)

    sums1 = sc_agg(xa, idx, zeros)
    h1a = tc1(sums1, xa, W_l1.T, b_l1[None, :], W_r1.T)
    sums2 = sc_agg(h1a, idx, zeros)
    out = tc2(sums2, h1a, W_l2.T, b_l2[None, :], W_r2.T)
    return out


# speedup vs baseline: 11.1983x; 1.3491x over previous
import functools

import jax
import jax.numpy as jnp
from jax import lax
from jax.experimental import pallas as pl
from jax.experimental.pallas import tpu as pltpu
from jax.experimental.pallas import tpu_sc as plsc

NC = 2
NS = 16
LN = 16
CH = 128
BLK = 8
RING = 2


def _pad_geometry(N, E):
    NW = NC * NS
    Np = -(-N // (NS * 8)) * (NS * 8)
    n_ch = -(-(-(-E // NW) // CH) // (2 * BLK)) * 2 * BLK
    return NW, Np, n_ch


def _make_sc_agg(N, E, D):
    NW, Np, n_ch = _pad_geometry(N, E)
    n_blk = n_ch // BLK
    pairs = n_blk // 2
    rows_per_tile = Np // NS

    mesh = plsc.VectorSubcoreMesh(core_axis_name="c", subcore_axis_name="s")

    @functools.partial(
        pl.kernel,
        mesh=mesh,
        out_type=jax.ShapeDtypeStruct((NC, Np, D), jnp.float32),
        scratch_types=[
            pltpu.VMEM((2, BLK, CH), jnp.int32),
            pltpu.VMEM((2, BLK, CH), jnp.int32),
            pltpu.VMEM((CH, D), jnp.float32),
            pltpu.VMEM((CH, D), jnp.float32),
            pltpu.VMEM_SHARED((Np, D), jnp.float32),
            pltpu.SemaphoreType.DMA,
            pltpu.SemaphoreType.DMA,
            pltpu.SemaphoreType.DMA,
            pltpu.SemaphoreType.DMA,
            pltpu.SemaphoreType.DMA,
            pltpu.SemaphoreType.DMA,
        ],
    )
    def agg(table_hbm, idx_hbm, zeros_hbm, out_hbm, ib0, ib1, r0, r1,
            accum_sh, gs0, gs1, ss0, ss1, is0, is1):
        c = lax.axis_index("c")
        s = lax.axis_index("s")
        wid = s * NC + c
        row0 = s * rows_per_tile
        rows = [r0, r1]
        gsem = [gs0, gs1]
        ssem = [ss0, ss1]

        pltpu.sync_copy(zeros_hbm.at[pl.ds(row0, rows_per_tile)],
                        accum_sh.at[pl.ds(row0, rows_per_tile)])
        plsc.subcore_barrier()

        def g_start(ib, k, sl):
            pltpu.async_copy(table_hbm.at[ib.at[0, k]], rows[sl], gsem[sl])

        def g_wait(sl):
            pltpu.make_async_copy(table_hbm.at[pl.ds(0, CH)], rows[sl],
                                  gsem[sl]).wait()

        def s_start(ib, k, sl):
            pltpu.async_copy(rows[sl], accum_sh.at[ib.at[1, k]], ssem[sl],
                             add=True)

        def s_wait(sl):
            pltpu.make_async_copy(table_hbm.at[pl.ds(0, CH)], rows[sl],
                                  ssem[sl]).wait()

        def i_start(b, ib, sem):
            pltpu.async_copy(idx_hbm.at[wid, b], ib, sem)

        def i_wait(ib, sem):
            pltpu.make_async_copy(idx_hbm.at[wid, 0], ib, sem).wait()

        def block(ib, waits_in, drain_out):
            for k in range(RING):
                if waits_in:
                    s_wait(k)
                g_start(ib, k, k)
            for k in range(BLK):
                sl = k % RING
                g_wait(sl)
                s_start(ib, k, sl)
                if k + RING < BLK:
                    s_wait(sl)
                    g_start(ib, k + RING, sl)
            if drain_out:
                for k in range(RING):
                    s_wait(k)

        i_start(0, ib0, is0)
        i_start(1, ib1, is1)

        def pair_body(p, carry):
            i_wait(ib0, is0)
            block(ib0, waits_in=False, drain_out=False)
            i_wait(ib1, is1)
            block(ib1, waits_in=True, drain_out=True)

            @pl.when(p < pairs - 1)
            def _():
                i_start(2 * p + 2, ib0, is0)
                i_start(2 * p + 3, ib1, is1)

            return carry

        lax.fori_loop(0, pairs, pair_body, 0)

        plsc.subcore_barrier()
        pltpu.sync_copy(accum_sh.at[pl.ds(row0, rows_per_tile)],
                        out_hbm.at[c, pl.ds(row0, rows_per_tile)])

    return agg


def _make_sc_count(N, E):
    NW, Np, n_ch = _pad_geometry(N, E)
    n_blk = n_ch // BLK
    NR = Np // CH
    NRp = -(-(NR + 1) // 8) * 8
    full = NRp // 8

    mesh = plsc.VectorSubcoreMesh(core_axis_name="c", subcore_axis_name="s")

    @functools.partial(
        pl.kernel,
        mesh=mesh,
        out_type=jax.ShapeDtypeStruct((NC, NRp, CH), jnp.float32),
        compiler_params=pltpu.CompilerParams(needs_layout_passes=False),
        scratch_types=[
            pltpu.VMEM((n_ch, CH), jnp.int32),
            pltpu.VMEM((NRp, CH), jnp.float32),
            pltpu.VMEM((NS, 8, CH), jnp.float32),
            pltpu.VMEM((8, CH), jnp.float32),
            pltpu.VMEM_SHARED((NS, NRp, CH), jnp.float32),
            pltpu.SemaphoreType.DMA,
        ],
    )
    def cnt_kernel(idx_hbm, out_hbm, dst_v, hist, sub, acc, stage_sh, sem):
        c = lax.axis_index("c")
        s = lax.axis_index("s")
        wid = s * NC + c
        zero16 = jnp.zeros((LN,), jnp.float32)
        ones16 = jnp.ones((LN,), jnp.float32)

        for b in range(n_blk):
            pltpu.sync_copy(idx_hbm.at[wid, b, 1],
                            dst_v.at[pl.ds(b * BLK, BLK)])

        def zero_row(r, carry):
            for q in range(CH // LN):
                hist[r, pl.ds(q * LN, LN)] = zero16
            return carry

        lax.fori_loop(0, NRp, zero_row, 0)

        def hist_row(b, carry):
            for q in range(CH // LN):
                idx16 = dst_v[b, pl.ds(q * LN, LN)]
                hi = lax.shift_right_logical(idx16, 7)
                lo = lax.bitwise_and(idx16, 127)
                plsc.addupdate_scatter(hist, [hi, lo], ones16)
            return carry

        lax.fori_loop(0, n_ch, hist_row, 0)

        pltpu.sync_copy(hist, stage_sh.at[s])
        plsc.subcore_barrier()

        def combine(nrows):
            r0 = s * 8
            pltpu.sync_copy(stage_sh.at[:, pl.ds(r0, nrows)],
                            sub.at[:, pl.ds(0, nrows)])

            def add_tile(t, carry):
                for r in range(nrows):
                    for q in range(CH // LN):
                        sl = pl.ds(q * LN, LN)
                        if t is None:
                            acc[r, sl] = sub[0, r, sl]
                        else:
                            acc[r, sl] = acc[r, sl] + sub[t, r, sl]
                return carry

            add_tile(None, 0)
            lax.fori_loop(1, NS, add_tile, 0)
            pltpu.sync_copy(acc.at[pl.ds(0, nrows)],
                            out_hbm.at[c, pl.ds(r0, nrows)])

        @pl.when(s < full)
        def _():
            combine(8)

    return cnt_kernel


def _make_tc_layer(N, Np, D, H, relu, R=1024):
    G = -(-N // R)

    def body(sums_ref, cnt_ref, x_ref, wlt_ref, bl_ref, wrt_ref, o_ref):
        sm = sums_ref[0] + sums_ref[1]
        cnt = jnp.maximum(cnt_ref[...], 1.0)
        agg = sm / cnt
        out = (jnp.dot(agg, wlt_ref[...], preferred_element_type=jnp.float32)
               + bl_ref[...]
               + jnp.dot(x_ref[...], wrt_ref[...],
                         preferred_element_type=jnp.float32))
        if relu:
            out = jnp.maximum(out, 0.0)
        o_ref[...] = out

    return pl.pallas_call(
        body,
        grid=(G,),
        in_specs=[
            pl.BlockSpec((NC, R, D), lambda i: (0, i, 0)),
            pl.BlockSpec((R, 1), lambda i: (i, 0)),
            pl.BlockSpec((R, D), lambda i: (i, 0)),
            pl.BlockSpec((D, H), lambda i: (0, 0)),
            pl.BlockSpec((1, H), lambda i: (0, 0)),
            pl.BlockSpec((D, H), lambda i: (0, 0)),
        ],
        out_specs=pl.BlockSpec((R, H), lambda i: (i, 0)),
        out_shape=jax.ShapeDtypeStruct((N, H), jnp.float32),
    )


@jax.jit
def kernel(x, edge_index, W_l1, b_l1, W_r1, W_l2, b_l2, W_r2):
    N, D = x.shape
    E = edge_index.shape[1]
    H = W_l1.shape[0]
    NW, Np, n_ch = _pad_geometry(N, E)
    n_blk = n_ch // BLK
    Ep = NW * n_ch * CH
    n_pad = Ep - E
    pad_src = jnp.arange(n_pad, dtype=jnp.int32) % N
    pad_dst = N + jnp.arange(n_pad, dtype=jnp.int32) % (Np - N)
    src = jnp.concatenate([edge_index[0], pad_src]).reshape(NW, n_blk, BLK, CH)
    dst = jnp.concatenate([edge_index[1], pad_dst]).reshape(NW, n_blk, BLK, CH)
    idx = jnp.stack([src, dst], axis=2)

    zeros = jnp.zeros((Np, D), dtype=jnp.float32)

    sc_agg = _make_sc_agg(N, E, D)
    sc_cnt = _make_sc_count(N, E)
    tc1 = _make_tc_layer(N, Np, D, H, relu=True)
    tc2 = _make_tc_layer(N, Np, H, H, relu=False)

    cnt_parts = sc_cnt(idx)
    cnt = (cnt_parts[0] + cnt_parts[1]).reshape(-1, 1)
    sums1 = sc_agg(x, idx, zeros)
    h1 = tc1(sums1, cnt, x, W_l1.T, b_l1[None, :], W_r1.T)
    sums2 = sc_agg(h1, idx, zeros)
    out = tc2(sums2, cnt, h1, W_l2.T, b_l2[None, :], W_r2.T)
    return out


# speedup vs baseline: 12.0356x; 1.0748x over previous
import functools

import jax
import jax.numpy as jnp
from jax import lax
from jax.experimental import pallas as pl
from jax.experimental.pallas import tpu as pltpu
from jax.experimental.pallas import tpu_sc as plsc

NC = 2
NS = 16
LN = 16
CH = 128
BLK = 8
RING = 2


def _pad_geometry(N, E):
    NW = NC * NS
    Np = -(-N // (NS * 8)) * (NS * 8)
    n_ch = -(-(-(-E // NW) // CH) // (2 * BLK)) * 2 * BLK
    return NW, Np, n_ch


def _make_sc_agg(N, E, D, with_hist):
    NW, Np, n_ch = _pad_geometry(N, E)
    n_blk = n_ch // BLK
    pairs = n_blk // 2
    rows_per_tile = Np // NS
    NR = Np // CH
    NRp = -(-(NR + 1) // 8) * 8

    mesh = plsc.VectorSubcoreMesh(core_axis_name="c", subcore_axis_name="s")

    out_type = [jax.ShapeDtypeStruct((NC, Np, D), jnp.float32)]
    scratch = [
        pltpu.VMEM((BLK, CH), jnp.int32),
        pltpu.VMEM((BLK, CH), jnp.int32),
        pltpu.VMEM((BLK, CH), jnp.int32),
        pltpu.VMEM((BLK, CH), jnp.int32),
        pltpu.VMEM((CH, D), jnp.float32),
        pltpu.VMEM((CH, D), jnp.float32),
        pltpu.VMEM_SHARED((Np, D), jnp.float32),
        pltpu.SemaphoreType.DMA,
        pltpu.SemaphoreType.DMA,
        pltpu.SemaphoreType.DMA,
        pltpu.SemaphoreType.DMA,
        pltpu.SemaphoreType.DMA,
        pltpu.SemaphoreType.DMA,
    ]
    if with_hist:
        out_type.append(jax.ShapeDtypeStruct((NW, NRp, CH), jnp.float32))
        scratch.append(pltpu.VMEM((NRp, CH), jnp.float32))

    @functools.partial(
        pl.kernel,
        mesh=mesh,
        out_type=tuple(out_type),
        compiler_params=pltpu.CompilerParams(needs_layout_passes=False),
        scratch_types=scratch,
    )
    def agg(table_hbm, src_hbm, dst_hbm, *rest):
        if with_hist:
            (out_hbm, hist_hbm, sib0, sib1, dib0, dib1, r0, r1, accum_sh,
             gs0, gs1, ss0, ss1, is0, is1, hist) = rest
        else:
            (out_hbm, sib0, sib1, dib0, dib1, r0, r1, accum_sh,
             gs0, gs1, ss0, ss1, is0, is1) = rest
            hist = None
        c = lax.axis_index("c")
        s = lax.axis_index("s")
        wid = s * NC + c
        row0 = s * rows_per_tile
        rows = [r0, r1]
        gsem = [gs0, gs1]
        ssem = [ss0, ss1]
        zero16 = jnp.zeros((LN,), jnp.float32)
        ones16 = jnp.ones((LN,), jnp.float32)

        def zero_row(r, carry):
            for q in range(D // LN):
                r0[r, pl.ds(q * LN, LN)] = zero16
            return carry

        lax.fori_loop(0, CH, zero_row, 0)
        n_full = rows_per_tile // CH
        for i in range(n_full):
            pltpu.sync_copy(r0, accum_sh.at[pl.ds(row0 + i * CH, CH)])
        tail = rows_per_tile - n_full * CH
        if tail:
            pltpu.sync_copy(r0.at[pl.ds(0, tail)],
                            accum_sh.at[pl.ds(row0 + n_full * CH, tail)])
        if with_hist:
            def zero_hrow(r, carry):
                for q in range(CH // LN):
                    hist[r, pl.ds(q * LN, LN)] = zero16
                return carry

            lax.fori_loop(0, NRp, zero_hrow, 0)
        plsc.subcore_barrier()

        def g_start(ib, k, sl):
            pltpu.async_copy(table_hbm.at[ib.at[k]], rows[sl], gsem[sl])

        def g_wait(sl):
            pltpu.make_async_copy(table_hbm.at[pl.ds(0, CH)], rows[sl],
                                  gsem[sl]).wait()

        def s_start(ib, k, sl):
            pltpu.async_copy(rows[sl], accum_sh.at[ib.at[k]], ssem[sl],
                             add=True)

        def s_wait(sl):
            pltpu.make_async_copy(table_hbm.at[pl.ds(0, CH)], rows[sl],
                                  ssem[sl]).wait()

        def i_start(b, sib, dib, sem):
            pltpu.async_copy(src_hbm.at[wid, b], sib, sem)
            pltpu.async_copy(dst_hbm.at[wid, b], dib, sem)

        def i_wait(sib, dib, sem):
            pltpu.make_async_copy(src_hbm.at[wid, 0], sib, sem).wait()
            pltpu.make_async_copy(dst_hbm.at[wid, 0], dib, sem).wait()

        def histo(dib):
            for k in range(BLK):
                for q in range(CH // LN):
                    idx16 = dib[k, pl.ds(q * LN, LN)]
                    hi = lax.shift_right_logical(idx16, 7)
                    lo = lax.bitwise_and(idx16, 127)
                    plsc.addupdate_scatter(hist, [hi, lo], ones16)

        def block(sib, dib, waits_in, drain_out):
            for k in range(RING):
                if waits_in:
                    s_wait(k)
                g_start(sib, k, k)
            if with_hist:
                histo(dib)
            for k in range(BLK):
                sl = k % RING
                g_wait(sl)
                s_start(dib, k, sl)
                if k + RING < BLK:
                    s_wait(sl)
                    g_start(sib, k + RING, sl)
            if drain_out:
                for k in range(RING):
                    s_wait(k)

        i_start(0, sib0, dib0, is0)
        i_start(1, sib1, dib1, is1)

        def pair_body(p, carry):
            i_wait(sib0, dib0, is0)
            block(sib0, dib0, waits_in=False, drain_out=False)
            i_wait(sib1, dib1, is1)
            block(sib1, dib1, waits_in=True, drain_out=True)

            @pl.when(p < pairs - 1)
            def _():
                i_start(2 * p + 2, sib0, dib0, is0)
                i_start(2 * p + 3, sib1, dib1, is1)

            return carry

        lax.fori_loop(0, pairs, pair_body, 0)

        plsc.subcore_barrier()
        pltpu.sync_copy(accum_sh.at[pl.ds(row0, rows_per_tile)],
                        out_hbm.at[c, pl.ds(row0, rows_per_tile)])
        if with_hist:
            pltpu.sync_copy(hist, hist_hbm.at[wid])

    return agg


def _make_tc_layer(N, Np, D, H, relu, R=1024):
    G = -(-N // R)

    def body(sums_ref, cnt_ref, x_ref, wlt_ref, bl_ref, wrt_ref, o_ref):
        sm = sums_ref[0] + sums_ref[1]
        cnt = jnp.maximum(cnt_ref[...], 1.0)
        agg = sm / cnt
        out = (jnp.dot(agg, wlt_ref[...], preferred_element_type=jnp.float32)
               + bl_ref[...]
               + jnp.dot(x_ref[...], wrt_ref[...],
                         preferred_element_type=jnp.float32))
        if relu:
            out = jnp.maximum(out, 0.0)
        o_ref[...] = out

    return pl.pallas_call(
        body,
        grid=(G,),
        in_specs=[
            pl.BlockSpec((NC, R, D), lambda i: (0, i, 0)),
            pl.BlockSpec((R, 1), lambda i: (i, 0)),
            pl.BlockSpec((R, D), lambda i: (i, 0)),
            pl.BlockSpec((D, H), lambda i: (0, 0)),
            pl.BlockSpec((1, H), lambda i: (0, 0)),
            pl.BlockSpec((D, H), lambda i: (0, 0)),
        ],
        out_specs=pl.BlockSpec((R, H), lambda i: (i, 0)),
        out_shape=jax.ShapeDtypeStruct((N, H), jnp.float32),
    )


@jax.jit
def kernel(x, edge_index, W_l1, b_l1, W_r1, W_l2, b_l2, W_r2):
    N, D = x.shape
    E = edge_index.shape[1]
    H = W_l1.shape[0]
    NW, Np, n_ch = _pad_geometry(N, E)
    n_blk = n_ch // BLK
    Ep = NW * n_ch * CH
    n_pad = Ep - E
    pad_src = jnp.arange(n_pad, dtype=jnp.int32) % N
    pad_dst = N + jnp.arange(n_pad, dtype=jnp.int32) % (Np - N)
    src = jnp.concatenate([edge_index[0], pad_src]).reshape(NW, n_blk, BLK, CH)
    dst = jnp.concatenate([edge_index[1], pad_dst]).reshape(NW, n_blk, BLK, CH)

    sc_agg1 = _make_sc_agg(N, E, D, with_hist=True)
    sc_agg2 = _make_sc_agg(N, E, H, with_hist=False)
    tc1 = _make_tc_layer(N, Np, D, H, relu=True)
    tc2 = _make_tc_layer(N, Np, H, H, relu=False)

    sums1, hist = sc_agg1(x, src, dst)
    cnt = jnp.sum(hist, axis=0).reshape(-1, 1)
    h1 = tc1(sums1, cnt, x, W_l1.T, b_l1[None, :], W_r1.T)
    (sums2,) = sc_agg2(h1, src, dst)
    out = tc2(sums2, cnt, h1, W_l2.T, b_l2[None, :], W_r2.T)
    return out
